# baseline (speedup 1.0000x reference)
import functools

import jax
import jax.numpy as jnp
from jax.experimental import pallas as pl
from jax.experimental.pallas import tpu as pltpu

B, S, D = 1, 2048, 1024
E, TOPK, DFF = 8, 2, 768
T = B * S
BT = 256


def _moe_body(hs_ref, gw_ref, guw_ref, dw_ref, out_ref, logits_ref, acc_ref):
    e = pl.program_id(0)
    t = pl.program_id(1)

    x = hs_ref[...]
    logits = jax.lax.dot_general(
        x, gw_ref[...], (((1,), (1,)), ((), ())),
        preferred_element_type=jnp.float32)
    logits_ref[...] = logits

    iota = jax.lax.broadcasted_iota(jnp.int32, (BT, E), 1)
    m1 = jnp.max(logits, axis=1, keepdims=True)
    idx1 = jnp.min(jnp.where(logits == m1, iota, E), axis=1, keepdims=True)
    l2 = jnp.where(iota == idx1, -jnp.inf, logits)
    m2 = jnp.max(l2, axis=1, keepdims=True)
    idx2 = jnp.min(jnp.where(l2 == m2, iota, E), axis=1, keepdims=True)
    p2 = jnp.exp(m2 - m1)
    denom = 1.0 + p2
    w1 = 1.0 / denom
    w2 = p2 / denom
    we = jnp.where(idx1 == e, w1, jnp.where(idx2 == e, w2, 0.0))

    xb = x.astype(jnp.bfloat16)
    gu = jax.lax.dot_general(
        xb, guw_ref[0], (((1,), (1,)), ((), ())),
        preferred_element_type=jnp.float32)
    g = gu[:, :DFF]
    u = gu[:, DFF:]
    act = g * jax.lax.logistic(g) * u
    y = jax.lax.dot_general(
        act.astype(jnp.bfloat16), dw_ref[0], (((1,), (1,)), ((), ())),
        preferred_element_type=jnp.float32)
    contrib = we * y

    sl = pl.ds(t * BT, BT)

    @pl.when(e == 0)
    def _():
        acc_ref[sl, :] = contrib

    @pl.when(e > 0)
    def _():
        acc_ref[sl, :] = acc_ref[sl, :] + contrib

    @pl.when(e == E - 1)
    def _():
        out_ref[...] = acc_ref[sl, :]


@functools.partial(jax.jit, static_argnums=())
def kernel(hidden_states, gate_w, gate_up_w, down_w):
    hs = hidden_states.reshape(T, D)
    guw_b = gate_up_w.astype(jnp.bfloat16)
    dw_b = down_w.astype(jnp.bfloat16)
    grid = (E, T // BT)
    out, logits = pl.pallas_call(
        _moe_body,
        grid=grid,
        in_specs=[
            pl.BlockSpec((BT, D), lambda e, t: (t, 0)),
            pl.BlockSpec((E, D), lambda e, t: (0, 0)),
            pl.BlockSpec((1, 2 * DFF, D), lambda e, t: (e, 0, 0)),
            pl.BlockSpec((1, D, DFF), lambda e, t: (e, 0, 0)),
        ],
        out_specs=[
            pl.BlockSpec((BT, D), lambda e, t: (t, 0)),
            pl.BlockSpec((BT, E), lambda e, t: (t, 0)),
        ],
        out_shape=[
            jax.ShapeDtypeStruct((T, D), jnp.float32),
            jax.ShapeDtypeStruct((T, E), jnp.float32),
        ],
        scratch_shapes=[pltpu.VMEM((T, D), jnp.float32)],
    )(hs, gate_w, guw_b, dw_b)
    return out.reshape(B, S, D), logits


# speedup vs baseline: 1.2203x; 1.2203x over previous
import functools

import jax
import jax.numpy as jnp
from jax import lax
from jax.experimental import pallas as pl
from jax.experimental.pallas import tpu as pltpu
from jax.experimental.pallas import tpu_sc as plsc

B, S, D = 1, 2048, 1024
E, TOPK, DFF = 8, 2, 768
T = B * S
P = T * TOPK

BT2 = 128
NTILES = P // BT2 + E
CAP = NTILES * BT2

NW = 32
TPW = T // NW


def _router_body(hs_ref, gw_ref, logits_ref, slot0_ref, slot1_ref,
                 w0row_ref, w1row_ref, counts_ref):
    x = hs_ref[...]
    logits = lax.dot_general(x, gw_ref[...], (((1,), (1,)), ((), ())),
                             preferred_element_type=jnp.float32)
    logits_ref[...] = logits

    iota = lax.broadcasted_iota(jnp.int32, (T, E), 1)
    m1 = jnp.max(logits, axis=1, keepdims=True)
    idx1 = jnp.min(jnp.where(logits == m1, iota, E), axis=1, keepdims=True)
    oh1 = iota == idx1
    l2 = jnp.where(oh1, -jnp.inf, logits)
    m2 = jnp.max(l2, axis=1, keepdims=True)
    idx2 = jnp.min(jnp.where(l2 == m2, iota, E), axis=1, keepdims=True)
    oh2 = iota == idx2
    p2 = jnp.exp(m2 - m1)
    denom = 1.0 + p2
    w0 = 1.0 / denom
    w1 = p2 / denom
    w0row_ref[...] = jnp.broadcast_to(w0, (T, 128))
    w1row_ref[...] = jnp.broadcast_to(w1, (T, 128))

    cnt = oh1.astype(jnp.int32) + oh2.astype(jnp.int32)
    inc = cnt
    s = 1
    while s < T:
        inc = inc + jnp.concatenate(
            [jnp.zeros((s, E), jnp.int32), inc[:T - s, :]], axis=0)
        s *= 2
    excl = inc - cnt
    counts = inc[T - 1:T, :]
    counts_ref[...] = counts

    tiles = (counts + (BT2 - 1)) // BT2
    tinc = tiles
    s = 1
    while s < E:
        tinc = tinc + jnp.concatenate(
            [jnp.zeros((1, s), jnp.int32), tinc[:, :E - s]], axis=1)
        s *= 2
    pad_off = (tinc - tiles) * BT2

    rank0 = jnp.sum(jnp.where(oh1, excl, 0), axis=1, keepdims=True)
    base0 = jnp.sum(jnp.where(oh1, jnp.broadcast_to(pad_off, (T, E)), 0),
                    axis=1, keepdims=True)
    rank1 = jnp.sum(jnp.where(oh2, excl, 0), axis=1, keepdims=True)
    base1 = jnp.sum(jnp.where(oh2, jnp.broadcast_to(pad_off, (T, E)), 0),
                    axis=1, keepdims=True)
    slot0_ref[...] = rank0 + base0
    slot1_ref[...] = rank1 + base1


def _run_router(hs, gate_w):
    return pl.pallas_call(
        _router_body,
        out_shape=[
            jax.ShapeDtypeStruct((T, E), jnp.float32),
            jax.ShapeDtypeStruct((T, 1), jnp.int32),
            jax.ShapeDtypeStruct((T, 1), jnp.int32),
            jax.ShapeDtypeStruct((T, 128), jnp.float32),
            jax.ShapeDtypeStruct((T, 128), jnp.float32),
            jax.ShapeDtypeStruct((1, E), jnp.int32),
        ],
    )(hs, gate_w)


def _make_dispatch():
    mesh = plsc.VectorSubcoreMesh(core_axis_name="c", subcore_axis_name="s")

    @functools.partial(
        pl.kernel, mesh=mesh,
        out_type=[
            jax.ShapeDtypeStruct((CAP, D), jnp.float32),
            jax.ShapeDtypeStruct((CAP, 128), jnp.float32),
        ],
        scratch_types=[
            pltpu.VMEM((TPW, D), jnp.float32),
            pltpu.VMEM((TPW,), jnp.int32),
            pltpu.VMEM((TPW,), jnp.int32),
            pltpu.VMEM((TPW, 128), jnp.float32),
            pltpu.VMEM((TPW, 128), jnp.float32),
            pltpu.SemaphoreType.DMA,
        ],
    )
    def dispatch(hs_hbm, slot0_hbm, slot1_hbm, w0row_hbm, w1row_hbm,
                 xs_hbm, ws_hbm, rows_v, idx0_v, idx1_v, wr0_v, wr1_v, sem):
        wid = lax.axis_index("s") * 2 + lax.axis_index("c")
        base = wid * TPW
        pltpu.sync_copy(hs_hbm.at[pl.ds(base, TPW), :], rows_v)
        pltpu.sync_copy(slot0_hbm.at[pl.ds(base, TPW)], idx0_v)
        pltpu.sync_copy(slot1_hbm.at[pl.ds(base, TPW)], idx1_v)
        pltpu.sync_copy(w0row_hbm.at[pl.ds(base, TPW), :], wr0_v)
        pltpu.sync_copy(w1row_hbm.at[pl.ds(base, TPW), :], wr1_v)
        c0 = pltpu.async_copy(rows_v, xs_hbm.at[idx0_v], sem)
        c1 = pltpu.async_copy(rows_v, xs_hbm.at[idx1_v], sem)
        c2 = pltpu.async_copy(wr0_v, ws_hbm.at[idx0_v], sem)
        c3 = pltpu.async_copy(wr1_v, ws_hbm.at[idx1_v], sem)
        c0.wait()
        c1.wait()
        c2.wait()
        c3.wait()

    return dispatch


def _ffn_body(te_ref, nu_ref, xs_ref, guw_ref, dw_ref, ws_ref, ys_ref):
    j = pl.program_id(0)

    @pl.when(j < nu_ref[0])
    def _():
        x = xs_ref[...]
        gu = lax.dot_general(x, guw_ref[0], (((1,), (1,)), ((), ())),
                             preferred_element_type=jnp.float32)
        g = gu[:, :DFF]
        u = gu[:, DFF:] * ws_ref[:, 0:1]
        act = g * lax.logistic(g) * u
        ys_ref[...] = lax.dot_general(act, dw_ref[0], (((1,), (1,)), ((), ())),
                                      preferred_element_type=jnp.float32)


def _run_ffn(te, nu, x_sorted, gate_up_w, down_w, wslot):
    grid_spec = pltpu.PrefetchScalarGridSpec(
        num_scalar_prefetch=2,
        grid=(NTILES,),
        in_specs=[
            pl.BlockSpec((BT2, D), lambda j, te, nu: (j, 0)),
            pl.BlockSpec((1, 2 * DFF, D), lambda j, te, nu: (te[j], 0, 0)),
            pl.BlockSpec((1, D, DFF), lambda j, te, nu: (te[j], 0, 0)),
            pl.BlockSpec((BT2, 128), lambda j, te, nu: (j, 0)),
        ],
        out_specs=pl.BlockSpec((BT2, D), lambda j, te, nu: (j, 0)),
    )
    return pl.pallas_call(
        _ffn_body,
        grid_spec=grid_spec,
        out_shape=jax.ShapeDtypeStruct((CAP, D), jnp.float32),
    )(te, nu, x_sorted, gate_up_w, down_w, wslot)


CH = 32


def _make_combine():
    mesh = plsc.VectorSubcoreMesh(core_axis_name="c", subcore_axis_name="s")

    @functools.partial(
        pl.kernel, mesh=mesh,
        out_type=jax.ShapeDtypeStruct((T, D), jnp.float32),
        scratch_types=[
            pltpu.VMEM((CH, D), jnp.float32),
            pltpu.VMEM((CH, D), jnp.float32),
            pltpu.VMEM((TPW,), jnp.int32),
            pltpu.VMEM((TPW,), jnp.int32),
            pltpu.SemaphoreType.DMA,
        ],
    )
    def combine(ys_hbm, slot0_hbm, slot1_hbm, out_hbm,
                a_v, b_v, idx0_v, idx1_v, sem):
        wid = lax.axis_index("s") * 2 + lax.axis_index("c")
        base = wid * TPW
        pltpu.sync_copy(slot0_hbm.at[pl.ds(base, TPW)], idx0_v)
        pltpu.sync_copy(slot1_hbm.at[pl.ds(base, TPW)], idx1_v)
        for ch in range(TPW // CH):
            c0 = pltpu.async_copy(ys_hbm.at[idx0_v.at[pl.ds(ch * CH, CH)]],
                                  a_v, sem)
            c1 = pltpu.async_copy(ys_hbm.at[idx1_v.at[pl.ds(ch * CH, CH)]],
                                  b_v, sem)
            c0.wait()
            c1.wait()

            def row(j, _):
                for q in range(D // 16):
                    sl = pl.ds(q * 16, 16)
                    a_v[j, sl] = a_v[j, sl] + b_v[j, sl]
                return 0

            lax.fori_loop(0, CH, row, 0)
            pltpu.sync_copy(a_v, out_hbm.at[pl.ds(base + ch * CH, CH), :])

    return combine


def kernel(hidden_states, gate_w, gate_up_w, down_w):
    hs = hidden_states.reshape(T, D)
    logits, slot0, slot1, w0row, w1row, counts = _run_router(hs, gate_w)

    counts_f = counts.reshape(E)
    tiles = (counts_f + (BT2 - 1)) // BT2
    starts = jnp.cumsum(tiles) - tiles
    nu = jnp.sum(tiles, dtype=jnp.int32).reshape(1)
    j_iota = jnp.arange(NTILES, dtype=jnp.int32)
    te = jnp.sum((j_iota[:, None] >= starts[None, :]).astype(jnp.int32),
                 axis=1) - 1
    te = jnp.clip(te, 0, E - 1)

    slot0_f = slot0.reshape(T)
    slot1_f = slot1.reshape(T)
    x_sorted, wslot = _make_dispatch()(hs, slot0_f, slot1_f, w0row, w1row)
    y_sorted = _run_ffn(te, nu, x_sorted, gate_up_w, down_w, wslot)
    out = _make_combine()(y_sorted, slot0_f, slot1_f)
    return out.reshape(B, S, D), logits
